# Initial kernel scaffold; baseline (speedup 1.0000x reference)
import jax
import jax.numpy as jnp
from jax.experimental import pallas as pl


def kernel(tokens, table, W, b, position):
    raise NotImplementedError("write your pallas kernel here")


# speedup vs baseline: 2.7390x; 2.7390x over previous
import functools

import jax
import jax.numpy as jnp
from jax import lax
from jax.experimental import pallas as pl
from jax.experimental.pallas import tpu as pltpu
from jax.experimental.pallas import tpu_sc as plsc

NC = 2
NS = 16
NW = NC * NS


def _sc_gather(table, idx):
    V, D = table.shape
    B = idx.shape[0]
    b_per_w = B // NW
    CH = 128
    n_ch = b_per_w // CH
    mesh = plsc.VectorSubcoreMesh(core_axis_name="c", subcore_axis_name="s")

    @functools.partial(
        pl.kernel,
        mesh=mesh,
        out_type=jax.ShapeDtypeStruct((B, D), jnp.float32),
        scratch_types=[
            pltpu.VMEM((b_per_w,), jnp.int32),
            pltpu.VMEM((CH, D), jnp.float32),
            pltpu.SemaphoreType.DMA,
        ],
    )
    def gather_kernel(table_hbm, idx_hbm, out_hbm, idx_v, rows_v, sem):
        wid = lax.axis_index("s") * NC + lax.axis_index("c")
        base = wid * b_per_w
        pltpu.sync_copy(idx_hbm.at[pl.ds(base, b_per_w)], idx_v)

        @pl.loop(0, n_ch)
        def _(c):
            off = c * CH
            pltpu.async_copy(
                table_hbm.at[idx_v.at[pl.ds(off, CH)]], rows_v, sem
            ).wait()
            pltpu.sync_copy(rows_v, out_hbm.at[pl.ds(base + off, CH)])

    return gather_kernel(table, idx)


def _tc_project(emb, W, b, position):
    B, D = emb.shape
    M = W.shape[0]
    L = position.shape[0]
    BLK = 512
    blocks_per_batch = L // BLK

    def mm_kernel(emb_ref, w_ref, b_ref, pos_ref, out_ref):
        acc = lax.dot_general(
            emb_ref[...], w_ref[...],
            dimension_numbers=(((1,), (1,)), ((), ())),
            preferred_element_type=jnp.float32,
        )
        out_ref[...] = acc + b_ref[...] + pos_ref[...]

    return pl.pallas_call(
        mm_kernel,
        grid=(B // BLK,),
        in_specs=[
            pl.BlockSpec((BLK, D), lambda i: (i, 0)),
            pl.BlockSpec((M, D), lambda i: (0, 0)),
            pl.BlockSpec((1, M), lambda i: (0, 0)),
            pl.BlockSpec((BLK, M), lambda i: (i % blocks_per_batch, 0)),
        ],
        out_specs=pl.BlockSpec((BLK, M), lambda i: (i, 0)),
        out_shape=jax.ShapeDtypeStruct((B, M), jnp.float32),
    )(emb, W, b.reshape(1, M), position)


def kernel(tokens, table, W, b, position):
    batch, seq = tokens.shape
    idx = tokens.reshape(-1).astype(jnp.int32)
    emb = _sc_gather(table, idx)
    out = _tc_project(emb, W, b, position)
    return out.reshape(batch, seq, W.shape[0])


# speedup vs baseline: 2.8313x; 1.0337x over previous
import functools

import jax
import jax.numpy as jnp
from jax import lax
from jax.experimental import pallas as pl
from jax.experimental.pallas import tpu as pltpu
from jax.experimental.pallas import tpu_sc as plsc

NC = 2
NS = 16
NW = NC * NS


def _sc_gather(table, idx):
    V, D = table.shape
    B = idx.shape[0]
    b_per_w = B // NW
    CH = 128
    n_ch = b_per_w // CH
    mesh = plsc.VectorSubcoreMesh(core_axis_name="c", subcore_axis_name="s")

    @functools.partial(
        pl.kernel,
        mesh=mesh,
        out_type=jax.ShapeDtypeStruct((B, D), jnp.float32),
        scratch_types=[
            pltpu.VMEM((b_per_w,), jnp.int32),
            pltpu.VMEM((CH, D), jnp.float32),
            pltpu.SemaphoreType.DMA,
        ],
    )
    def gather_kernel(table_hbm, idx_hbm, out_hbm, idx_v, rows_v, sem):
        wid = lax.axis_index("s") * NC + lax.axis_index("c")
        base = wid * b_per_w
        pltpu.sync_copy(idx_hbm.at[pl.ds(base, b_per_w)], idx_v)

        @pl.loop(0, n_ch)
        def _(c):
            off = c * CH
            pltpu.async_copy(
                table_hbm.at[idx_v.at[pl.ds(off, CH)]], rows_v, sem
            ).wait()
            pltpu.sync_copy(rows_v, out_hbm.at[pl.ds(base + off, CH)])

    return gather_kernel(table, idx)


def _tc_project(emb, W, b, position):
    B, D = emb.shape
    M = W.shape[0]
    L = position.shape[0]
    BLK = 512
    j_blocks = L // BLK
    k_blocks = B // L

    def mm_kernel(emb_ref, w_ref, b_ref, pos_ref, out_ref):
        acc = lax.dot_general(
            emb_ref[...].astype(jnp.bfloat16), w_ref[...],
            dimension_numbers=(((1,), (1,)), ((), ())),
            preferred_element_type=jnp.float32,
        )
        out_ref[...] = acc + b_ref[...] + pos_ref[...]

    return pl.pallas_call(
        mm_kernel,
        grid=(j_blocks, k_blocks),
        in_specs=[
            pl.BlockSpec((BLK, D), lambda j, k: (k * j_blocks + j, 0)),
            pl.BlockSpec((M, D), lambda j, k: (0, 0)),
            pl.BlockSpec((1, M), lambda j, k: (0, 0)),
            pl.BlockSpec((BLK, M), lambda j, k: (j, 0)),
        ],
        out_specs=pl.BlockSpec((BLK, M), lambda j, k: (k * j_blocks + j, 0)),
        out_shape=jax.ShapeDtypeStruct((B, M), jnp.float32),
    )(emb, W.astype(jnp.bfloat16), b.reshape(1, M), position)


def kernel(tokens, table, W, b, position):
    batch, seq = tokens.shape
    idx = tokens.reshape(-1).astype(jnp.int32)
    emb = _sc_gather(table, idx)
    out = _tc_project(emb, W, b, position)
    return out.reshape(batch, seq, W.shape[0])
